# Initial kernel scaffold; baseline (speedup 1.0000x reference)
import jax
import jax.numpy as jnp
from jax.experimental import pallas as pl


def kernel(atten_map, gt_bboxes):
    raise NotImplementedError("write your pallas kernel here")


# speedup vs baseline: 64.6971x; 64.6971x over previous
import numpy as np
import jax
import jax.numpy as jnp
from jax.experimental import pallas as pl
from jax.experimental.pallas import tpu as pltpu

_H = 40
_W = 40
_HW = _H * _W
_C = 512
_PC_LO_X = -51.2
_PC_LO_Y = -51.2
_DIM_X = 102.4
_DIM_Y = 102.4
_CELL_X = np.float32(_DIM_X / _W)
_CELL_Y = np.float32(_DIM_Y / _H)
_RATIO_LO = 1.0
_RATIO_HI = 6.0

_ww, _hh = np.meshgrid(range(_W), range(_H))
_wwf = (_ww.reshape(-1).astype(np.float64) + 0.5) / _W * _DIM_X + _PC_LO_X
_hhf = (_hh.reshape(-1).astype(np.float64) + 0.5) / _H * _DIM_Y + _PC_LO_Y
_GRIDS = np.stack([_wwf, _hhf], 1).astype(np.float32)


def _loss_body(a_ref, g_ref, gr_ref, out_ref, acc_ref):
    b = pl.program_id(0)

    @pl.when(b == 0)
    def _init():
        acc_ref[0] = 0.0
        acc_ref[1] = 0.0

    x = a_ref[0]
    s1 = jnp.sum(x, axis=1, keepdims=True)
    s2 = jnp.sum(x * x, axis=1, keepdims=True)
    v = (s2 - s1 * s1 * (1.0 / _C)) * (1.0 / (_C - 1))

    g = g_ref[0]
    cx = g[0:1]
    cy = g[1:2]
    dl = g[3:4]
    dw = g[4:5]
    yaw = g[6:7]
    rl = jnp.clip(_CELL_X / dl, _RATIO_LO, _RATIO_HI)
    rw = jnp.clip(_CELL_Y / dw, _RATIO_LO, _RATIO_HI)
    hx = 0.5 * dl * rl
    hy = 0.5 * dw * rw
    sn = jnp.sin(yaw)
    cs = jnp.cos(yaw)
    xs = []
    ys = []
    for sx, sy in ((-1.0, -1.0), (-1.0, 1.0), (1.0, 1.0), (1.0, -1.0)):
        lx = sx * hx
        ly = sy * hy
        xs.append(lx * cs - ly * sn + cx)
        ys.append(lx * sn + ly * cs + cy)

    gx = gr_ref[:, 0:1]
    gy = gr_ref[:, 1:2]
    all_ge = None
    all_le = None
    for k in range(4):
        kn = (k + 1) % 4
        ex = xs[kn] - xs[k]
        ey = ys[kn] - ys[k]
        cross = ex * (gy - ys[k]) - ey * (gx - xs[k])
        ge = cross >= 0.0
        le = cross <= 0.0
        all_ge = ge if all_ge is None else (all_ge & ge)
        all_le = le if all_le is None else (all_le & le)
    inside = all_ge | all_le

    d2 = (gx - cx) ** 2 + (gy - cy) ** 2
    mind = jnp.min(d2, axis=0, keepdims=True)
    cellid = jax.lax.broadcasted_iota(jnp.int32, (_HW, 32), 0)
    cand = jnp.where(d2 == mind, cellid, _HW)
    mi = jnp.min(cand, axis=0, keepdims=True)
    pos = inside | (cellid == mi)

    ki = jnp.sum(pos.astype(jnp.int32), axis=1, keepdims=True,
                 dtype=jnp.int32)
    boxid = jax.lax.broadcasted_iota(jnp.int32, (_HW, 32), 1)
    lastc = jnp.max(jnp.where(pos, boxid, -1), axis=1, keepdims=True)
    own = ((ki & 1) == 1) & (boxid == lastc)

    ownf = own.astype(jnp.float32)
    cnt = jnp.sum(ownf, axis=0, keepdims=True)
    vs = jnp.sum(ownf * v, axis=0, keepdims=True)
    has = cnt > 0.0
    contrib = jnp.where(has, vs / jnp.maximum(cnt, 1.0), 0.0)
    acc_ref[0] += -jnp.sum(contrib)
    acc_ref[1] += jnp.sum(has.astype(jnp.float32))

    @pl.when(b == pl.num_programs(0) - 1)
    def _fin():
        out_ref[0, 0] = acc_ref[0] / jnp.maximum(acc_ref[1], 1.0)


def kernel(atten_map, gt_bboxes):
    B = atten_map.shape[0]
    gtT = jnp.transpose(gt_bboxes.astype(jnp.float32), (0, 2, 1))
    grids = jnp.asarray(_GRIDS)
    out = pl.pallas_call(
        _loss_body,
        grid=(B,),
        in_specs=[
            pl.BlockSpec((1, _HW, _C), lambda b: (b, b * 0, b * 0)),
            pl.BlockSpec((1, 7, 32), lambda b: (b, b * 0, b * 0)),
            pl.BlockSpec((_HW, 2), lambda b: (b * 0, b * 0)),
        ],
        out_specs=pl.BlockSpec((1, 1), lambda b: (b * 0, b * 0),
                               memory_space=pltpu.SMEM),
        out_shape=jax.ShapeDtypeStruct((1, 1), jnp.float32),
        scratch_shapes=[pltpu.SMEM((2,), jnp.float32)],
    )(atten_map, gtT, grids)
    return out[0, 0]
